# Initial kernel scaffold; baseline (speedup 1.0000x reference)
import jax
import jax.numpy as jnp
from jax.experimental import pallas as pl


def kernel(x, edge_index, batch, W0, b0, W1, b1, W2, b2, bias):
    raise NotImplementedError("write your pallas kernel here")


# speedup vs baseline: 3.8290x; 3.8290x over previous
import functools

import jax
import jax.numpy as jnp
from jax import lax
from jax.experimental import pallas as pl
from jax.experimental.pallas import tpu as pltpu
from jax.experimental.pallas import tpu_sc as plsc

N = 10000
E = 320000
D = 128
H = 64
T = 4
L = 3
G = 512

NC, NS = 2, 16
CH = 128
NP_ = 10240
EP = 327680
GP = 640
EB_A = EP // (NC * NS) // CH
EB_L = EP // NS // CH
RB = NP_ // NS // CH
RBP = 8
RG = GP // NS

_mesh = plsc.VectorSubcoreMesh(core_axis_name="c", subcore_axis_name="s",
                               num_cores=NC, num_subcores=NS)


SB = 40


def _zero_stripes(zeros_hbm, zbuf, acc, racc, s):
    pltpu.sync_copy(zeros_hbm, zbuf)

    def body(k, _):
        pltpu.sync_copy(zbuf, acc.at[pl.ds((s * RB + k) * CH, CH)])
        return _

    lax.fori_loop(0, RB, body, None)
    if racc is not None:
        pltpu.sync_copy(zbuf.at[pl.ds(0, RG)], racc.at[pl.ds(s * RG, RG)])


def _edge_loop(table_hbm, src_hbm, dst_hbm, base, idx_s, idx_d, rows, sem, acc,
               nblocks):
    def inner(j, _):
        pltpu.async_copy(table_hbm.at[idx_s.at[j]], rows, sem).wait()
        pltpu.sync_copy(rows, acc.at[idx_d.at[j]], add=True)
        return _

    def outer(ob, _):
        pltpu.sync_copy(src_hbm.at[pl.ds(base + ob * SB, SB)], idx_s)
        pltpu.sync_copy(dst_hbm.at[pl.ds(base + ob * SB, SB)], idx_d)
        lax.fori_loop(0, SB, inner, None)
        return _

    lax.fori_loop(0, nblocks // SB, outer, None)


def _readout_loop(h_hbm, batch_hbm, bidx, rows, racc, c, s):
    pltpu.sync_copy(batch_hbm.at[pl.ds(s * RBP, RBP)], bidx)

    def body(k, _):
        pltpu.sync_copy(h_hbm.at[pl.ds(c * NP_ + (s * RB + k) * CH, CH)], rows)
        pltpu.sync_copy(rows, racc.at[bidx.at[k]], add=True)
        return _

    lax.fori_loop(0, RB, body, None)


def _copyout_acc(acc, rows, out_hbm, c, s):
    def body(k, _):
        r0 = (s * RB + k) * CH
        pltpu.sync_copy(acc.at[pl.ds(r0, CH)], rows)
        pltpu.sync_copy(rows, out_hbm.at[c, pl.ds(r0, CH)])
        return _

    lax.fori_loop(0, RB, body, None)


def _copyout_racc(racc, rows, r_hbm, c, s):
    pltpu.sync_copy(racc.at[pl.ds(s * RG, RG)], rows.at[pl.ds(0, RG)])
    pltpu.sync_copy(rows.at[pl.ds(0, RG)], r_hbm.at[c, pl.ds(s * RG, RG)])


@functools.partial(
    pl.kernel,
    out_type=jax.ShapeDtypeStruct((NC, NP_, D), jnp.float32),
    mesh=_mesh,
    scratch_types=[
        pltpu.VMEM_SHARED((NP_, D), jnp.float32),
        pltpu.VMEM((SB, CH), jnp.int32),
        pltpu.VMEM((SB, CH), jnp.int32),
        pltpu.VMEM((CH, D), jnp.float32),
        pltpu.SemaphoreType.DMA,
    ],
)
def _sc_agg_x(x_hbm, src_hbm, dst_hbm, zeros_hbm, out_hbm,
              acc, idx_s, idx_d, rows, sem):
    c = lax.axis_index("c")
    s = lax.axis_index("s")
    tid = c * NS + s
    _zero_stripes(zeros_hbm, rows, acc, None, s)
    plsc.subcore_barrier()
    _edge_loop(x_hbm, src_hbm, dst_hbm, tid * EB_A, idx_s, idx_d, rows, sem,
               acc, EB_A)
    plsc.subcore_barrier()
    _copyout_acc(acc, rows, out_hbm, c, s)


@functools.partial(
    pl.kernel,
    out_type=(jax.ShapeDtypeStruct((NC, NP_, D), jnp.float32),
              jax.ShapeDtypeStruct((NC, GP, D), jnp.float32)),
    mesh=_mesh,
    scratch_types=[
        pltpu.VMEM_SHARED((NP_, D), jnp.float32),
        pltpu.VMEM_SHARED((GP, D), jnp.float32),
        pltpu.VMEM((SB, CH), jnp.int32),
        pltpu.VMEM((SB, CH), jnp.int32),
        pltpu.VMEM((RBP, CH), jnp.int32),
        pltpu.VMEM((CH, D), jnp.float32),
        pltpu.SemaphoreType.DMA,
    ],
)
def _sc_agg_h(h_hbm, srcb_hbm, dstd_hbm, batch_hbm, zeros_hbm, agg_hbm, r_hbm,
              acc, racc, idx_s, idx_d, bidx, rows, sem):
    c = lax.axis_index("c")
    s = lax.axis_index("s")
    _zero_stripes(zeros_hbm, rows, acc, racc, s)
    plsc.subcore_barrier()
    base = c * (EP // CH) + s * EB_L
    _edge_loop(h_hbm, srcb_hbm, dstd_hbm, base, idx_s, idx_d, rows, sem,
               acc, EB_L)
    _readout_loop(h_hbm, batch_hbm, bidx, rows, racc, c, s)
    plsc.subcore_barrier()
    _copyout_acc(acc, rows, agg_hbm, c, s)
    _copyout_racc(racc, rows, r_hbm, c, s)


@functools.partial(
    pl.kernel,
    out_type=jax.ShapeDtypeStruct((NC, GP, D), jnp.float32),
    mesh=_mesh,
    scratch_types=[
        pltpu.VMEM_SHARED((GP, D), jnp.float32),
        pltpu.VMEM((RBP, CH), jnp.int32),
        pltpu.VMEM((CH, D), jnp.float32),
    ],
)
def _sc_readout(h_hbm, batch_hbm, zeros_hbm, r_hbm, racc, bidx, rows):
    c = lax.axis_index("c")
    s = lax.axis_index("s")
    pltpu.sync_copy(zeros_hbm, rows)
    pltpu.sync_copy(rows.at[pl.ds(0, RG)], racc.at[pl.ds(s * RG, RG)])
    plsc.subcore_barrier()
    _readout_loop(h_hbm, batch_hbm, bidx, rows, racc, c, s)
    plsc.subcore_barrier()
    _copyout_racc(racc, rows, r_hbm, c, s)


_BN = 256


def _tc1_body(x_ref, a_ref, b_ref, w_ref, bias_ref, o_ref):
    u = x_ref[...] + a_ref[0] + b_ref[0]
    h = jnp.dot(u, w_ref[...], preferred_element_type=jnp.float32)
    o_ref[0] = jnp.maximum(h + bias_ref[0], 0.0)


def _tc1(x_pad, agg0p, w_cat, b_cat):
    return pl.pallas_call(
        _tc1_body,
        grid=(NC, NP_ // _BN),
        in_specs=[
            pl.BlockSpec((_BN, D), lambda h, i: (i, 0)),
            pl.BlockSpec((1, _BN, D), lambda h, i: (0, i, 0)),
            pl.BlockSpec((1, _BN, D), lambda h, i: (1, i, 0)),
            pl.BlockSpec((D, D), lambda h, i: (0, h)),
            pl.BlockSpec((1, 1, D), lambda h, i: (h, 0, 0)),
        ],
        out_specs=pl.BlockSpec((1, _BN, D), lambda h, i: (h, i, 0)),
        out_shape=jax.ShapeDtypeStruct((NC, NP_, D), jnp.float32),
    )(x_pad, agg0p, agg0p, w_cat, b_cat)


def _tc_mlp_body(v_ref, a_ref, w_ref, bias_ref, o_ref):
    u = v_ref[0] + a_ref[0]
    h = jnp.dot(u, w_ref[0], preferred_element_type=jnp.float32)
    o_ref[0] = jnp.maximum(h + bias_ref[0], 0.0)


def _tc_mlp(h_prev, agg, w_quad, b_cat):
    return pl.pallas_call(
        _tc_mlp_body,
        grid=(NC, NP_ // _BN),
        in_specs=[
            pl.BlockSpec((1, _BN, D), lambda h, i: (h, i, 0)),
            pl.BlockSpec((1, _BN, D), lambda h, i: (h, i, 0)),
            pl.BlockSpec((1, D, D), lambda h, i: (h, 0, 0)),
            pl.BlockSpec((1, 1, D), lambda h, i: (h, 0, 0)),
        ],
        out_specs=pl.BlockSpec((1, _BN, D), lambda h, i: (h, i, 0)),
        out_shape=jax.ShapeDtypeStruct((NC, NP_, D), jnp.float32),
    )(h_prev, agg, w_quad, b_cat)


def _quad(W):
    q = jnp.zeros((NC, 2 * H, 2 * H), jnp.float32)
    q = q.at[:, :H, :H].set(W[0::2])
    q = q.at[:, H:, H:].set(W[1::2])
    return q


def _fold(r):
    return jnp.concatenate([r[0, :G], r[1, :G]], axis=1).reshape(G, T, H)


def kernel(x, edge_index, batch, W0, b0, W1, b1, W2, b2, bias):
    src = edge_index[0]
    dst = edge_index[1]
    srcp = jnp.full((EP,), NP_ - 1, jnp.int32).at[:E].set(src)
    dstp = jnp.full((EP,), NP_ - 1, jnp.int32).at[:E].set(dst).reshape(EP // CH, CH)
    srcb = jnp.stack([srcp, srcp + NP_]).reshape(NC * (EP // CH), CH)
    dstd = jnp.concatenate([dstp, dstp], axis=0)
    srcp = srcp.reshape(EP // CH, CH)
    batchp = jnp.full((NS, RBP, CH), G, jnp.int32).at[:, :RB].set(
        jnp.full((NP_,), G, jnp.int32).at[:N].set(batch)
        .reshape(NS, RB, CH)).reshape(NS * RBP, CH)
    x_pad = jnp.zeros((NP_, D), jnp.float32).at[:N].set(x)
    zeros128 = jnp.zeros((CH, CH), jnp.float32)

    w0_cat = jnp.moveaxis(W0, 0, 1).reshape(D, T * H)
    b0_cat = b0.reshape(NC, 1, D)
    w1_quad, b1_cat = _quad(W1), b1.reshape(NC, 1, D)
    w2_quad, b2_cat = _quad(W2), b2.reshape(NC, 1, D)

    agg0p = _sc_agg_x(x_pad, srcp, dstp, zeros128)
    h1 = _tc1(x_pad, agg0p, w0_cat, b0_cat)
    h1f = h1.reshape(NC * NP_, D)
    agg1, r1 = _sc_agg_h(h1f, srcb, dstd, batchp, zeros128)
    h2 = _tc_mlp(h1, agg1, w1_quad, b1_cat)
    h2f = h2.reshape(NC * NP_, D)
    agg2, r2 = _sc_agg_h(h2f, srcb, dstd, batchp, zeros128)
    h3 = _tc_mlp(h2, agg2, w2_quad, b2_cat)
    r3 = _sc_readout(h3.reshape(NC * NP_, D), batchp, zeros128)

    out = jnp.concatenate([_fold(r1), _fold(r2), _fold(r3)], axis=2)
    return out.at[:, 0, :].add(bias[0, 0])


# speedup vs baseline: 4.3731x; 1.1421x over previous
import functools

import jax
import jax.numpy as jnp
from jax import lax
from jax.experimental import pallas as pl
from jax.experimental.pallas import tpu as pltpu
from jax.experimental.pallas import tpu_sc as plsc

N = 10000
E = 320000
D = 128
H = 64
T = 4
L = 3
G = 512

NC, NS = 2, 16
CH = 128
NP_ = 10240
EP = 327680
GP = 640
EB_A = EP // (NC * NS) // CH
EB_L = EP // NS // CH
RB = NP_ // NS // CH
RBP = 8
RG = GP // NS

_mesh = plsc.VectorSubcoreMesh(core_axis_name="c", subcore_axis_name="s",
                               num_cores=NC, num_subcores=NS)


SB = 16


def _zero_stripes(zeros_hbm, zbuf, acc, racc, s):
    pltpu.sync_copy(zeros_hbm, zbuf)

    def body(k, _):
        pltpu.sync_copy(zbuf, acc.at[pl.ds((s * RB + k) * CH, CH)])
        return _

    lax.fori_loop(0, RB, body, None)
    if racc is not None:
        pltpu.sync_copy(zbuf.at[pl.ds(0, RG)], racc.at[pl.ds(s * RG, RG)])


def _edge_loop(table_hbm, src_hbm, dst_hbm, base, idx_s, idx_d,
               rows0, rows1, sem0, sem1, acc, nblocks):
    def gather(c, rows, sem):
        pltpu.async_copy(table_hbm.at[idx_s.at[c]], rows, sem)

    def gwait(c, rows, sem):
        pltpu.make_async_copy(table_hbm.at[idx_s.at[c]], rows, sem).wait()

    def scatter(c, rows):
        pltpu.sync_copy(rows, acc.at[idx_d.at[c]], add=True)

    def outer(ob, _):
        pltpu.sync_copy(src_hbm.at[pl.ds(base + ob * SB, SB)], idx_s)
        pltpu.sync_copy(dst_hbm.at[pl.ds(base + ob * SB, SB)], idx_d)
        gather(0, rows0, sem0)
        gather(1, rows1, sem1)

        def pair(p, _):
            gwait(2 * p, rows0, sem0)
            scatter(2 * p, rows0)
            gather(2 * p + 2, rows0, sem0)
            gwait(2 * p + 1, rows1, sem1)
            scatter(2 * p + 1, rows1)
            gather(2 * p + 3, rows1, sem1)
            return _

        lax.fori_loop(0, SB // 2 - 1, pair, None)
        gwait(SB - 2, rows0, sem0)
        scatter(SB - 2, rows0)
        gwait(SB - 1, rows1, sem1)
        scatter(SB - 1, rows1)
        return _

    lax.fori_loop(0, nblocks // SB, outer, None)


def _readout_loop(h_hbm, batch_hbm, bidx, rows, racc, c, s):
    pltpu.sync_copy(batch_hbm.at[pl.ds(s * RBP, RBP)], bidx)

    def body(k, _):
        pltpu.sync_copy(h_hbm.at[pl.ds(c * NP_ + (s * RB + k) * CH, CH)], rows)
        pltpu.sync_copy(rows, racc.at[bidx.at[k]], add=True)
        return _

    lax.fori_loop(0, RB, body, None)


def _copyout_acc(acc, rows, out_hbm, c, s):
    def body(k, _):
        r0 = (s * RB + k) * CH
        pltpu.sync_copy(acc.at[pl.ds(r0, CH)], rows)
        pltpu.sync_copy(rows, out_hbm.at[c, pl.ds(r0, CH)])
        return _

    lax.fori_loop(0, RB, body, None)


def _copyout_racc(racc, rows, r_hbm, c, s):
    pltpu.sync_copy(racc.at[pl.ds(s * RG, RG)], rows.at[pl.ds(0, RG)])
    pltpu.sync_copy(rows.at[pl.ds(0, RG)], r_hbm.at[c, pl.ds(s * RG, RG)])


@functools.partial(
    pl.kernel,
    out_type=jax.ShapeDtypeStruct((NC, NP_, D), jnp.float32),
    mesh=_mesh,
    scratch_types=[
        pltpu.VMEM_SHARED((NP_, D), jnp.float32),
        pltpu.VMEM((SB, CH), jnp.int32),
        pltpu.VMEM((SB, CH), jnp.int32),
        pltpu.VMEM((CH, D), jnp.float32),
        pltpu.VMEM((CH, D), jnp.float32),
        pltpu.SemaphoreType.DMA,
        pltpu.SemaphoreType.DMA,
    ],
)
def _sc_agg_x(x_hbm, src_hbm, dst_hbm, zeros_hbm, out_hbm,
              acc, idx_s, idx_d, rows, rows1, sem, sem1):
    c = lax.axis_index("c")
    s = lax.axis_index("s")
    tid = c * NS + s
    _zero_stripes(zeros_hbm, rows, acc, None, s)
    plsc.subcore_barrier()
    _edge_loop(x_hbm, src_hbm, dst_hbm, tid * EB_A, idx_s, idx_d,
               rows, rows1, sem, sem1, acc, EB_A)
    plsc.subcore_barrier()
    _copyout_acc(acc, rows, out_hbm, c, s)


@functools.partial(
    pl.kernel,
    out_type=(jax.ShapeDtypeStruct((NC, NP_, D), jnp.float32),
              jax.ShapeDtypeStruct((NC, GP, D), jnp.float32)),
    mesh=_mesh,
    scratch_types=[
        pltpu.VMEM_SHARED((NP_, D), jnp.float32),
        pltpu.VMEM_SHARED((GP, D), jnp.float32),
        pltpu.VMEM((SB, CH), jnp.int32),
        pltpu.VMEM((SB, CH), jnp.int32),
        pltpu.VMEM((RBP, CH), jnp.int32),
        pltpu.VMEM((CH, D), jnp.float32),
        pltpu.VMEM((CH, D), jnp.float32),
        pltpu.SemaphoreType.DMA,
        pltpu.SemaphoreType.DMA,
    ],
)
def _sc_agg_h(h_hbm, srcb_hbm, dstd_hbm, batch_hbm, zeros_hbm, agg_hbm, r_hbm,
              acc, racc, idx_s, idx_d, bidx, rows, rows1, sem, sem1):
    c = lax.axis_index("c")
    s = lax.axis_index("s")
    _zero_stripes(zeros_hbm, rows, acc, racc, s)
    plsc.subcore_barrier()
    base = c * (EP // CH) + s * EB_L
    _edge_loop(h_hbm, srcb_hbm, dstd_hbm, base, idx_s, idx_d,
               rows, rows1, sem, sem1, acc, EB_L)
    _readout_loop(h_hbm, batch_hbm, bidx, rows, racc, c, s)
    plsc.subcore_barrier()
    _copyout_acc(acc, rows, agg_hbm, c, s)
    _copyout_racc(racc, rows, r_hbm, c, s)


@functools.partial(
    pl.kernel,
    out_type=jax.ShapeDtypeStruct((NC, GP, D), jnp.float32),
    mesh=_mesh,
    scratch_types=[
        pltpu.VMEM_SHARED((GP, D), jnp.float32),
        pltpu.VMEM((RBP, CH), jnp.int32),
        pltpu.VMEM((CH, D), jnp.float32),
    ],
)
def _sc_readout(h_hbm, batch_hbm, zeros_hbm, r_hbm, racc, bidx, rows):
    c = lax.axis_index("c")
    s = lax.axis_index("s")
    pltpu.sync_copy(zeros_hbm, rows)
    pltpu.sync_copy(rows.at[pl.ds(0, RG)], racc.at[pl.ds(s * RG, RG)])
    plsc.subcore_barrier()
    _readout_loop(h_hbm, batch_hbm, bidx, rows, racc, c, s)
    plsc.subcore_barrier()
    _copyout_racc(racc, rows, r_hbm, c, s)


_BN = 256


def _tc1_body(x_ref, a_ref, b_ref, w_ref, bias_ref, o_ref):
    u = x_ref[...] + a_ref[0] + b_ref[0]
    h = jnp.dot(u, w_ref[...], preferred_element_type=jnp.float32)
    o_ref[0] = jnp.maximum(h + bias_ref[0], 0.0)


def _tc1(x_pad, agg0p, w_cat, b_cat):
    return pl.pallas_call(
        _tc1_body,
        grid=(NC, NP_ // _BN),
        in_specs=[
            pl.BlockSpec((_BN, D), lambda h, i: (i, 0)),
            pl.BlockSpec((1, _BN, D), lambda h, i: (0, i, 0)),
            pl.BlockSpec((1, _BN, D), lambda h, i: (1, i, 0)),
            pl.BlockSpec((D, D), lambda h, i: (0, h)),
            pl.BlockSpec((1, 1, D), lambda h, i: (h, 0, 0)),
        ],
        out_specs=pl.BlockSpec((1, _BN, D), lambda h, i: (h, i, 0)),
        out_shape=jax.ShapeDtypeStruct((NC, NP_, D), jnp.float32),
    )(x_pad, agg0p, agg0p, w_cat, b_cat)


def _tc_mlp_body(v_ref, a_ref, w_ref, bias_ref, o_ref):
    u = v_ref[0] + a_ref[0]
    h = jnp.dot(u, w_ref[0], preferred_element_type=jnp.float32)
    o_ref[0] = jnp.maximum(h + bias_ref[0], 0.0)


def _tc_mlp(h_prev, agg, w_quad, b_cat):
    return pl.pallas_call(
        _tc_mlp_body,
        grid=(NC, NP_ // _BN),
        in_specs=[
            pl.BlockSpec((1, _BN, D), lambda h, i: (h, i, 0)),
            pl.BlockSpec((1, _BN, D), lambda h, i: (h, i, 0)),
            pl.BlockSpec((1, D, D), lambda h, i: (h, 0, 0)),
            pl.BlockSpec((1, 1, D), lambda h, i: (h, 0, 0)),
        ],
        out_specs=pl.BlockSpec((1, _BN, D), lambda h, i: (h, i, 0)),
        out_shape=jax.ShapeDtypeStruct((NC, NP_, D), jnp.float32),
    )(h_prev, agg, w_quad, b_cat)


def _quad(W):
    q = jnp.zeros((NC, 2 * H, 2 * H), jnp.float32)
    q = q.at[:, :H, :H].set(W[0::2])
    q = q.at[:, H:, H:].set(W[1::2])
    return q


def _fold(r):
    return jnp.concatenate([r[0, :G], r[1, :G]], axis=1).reshape(G, T, H)


def kernel(x, edge_index, batch, W0, b0, W1, b1, W2, b2, bias):
    src = edge_index[0]
    dst = edge_index[1]
    srcp = jnp.full((EP,), NP_ - 1, jnp.int32).at[:E].set(src)
    dstp = jnp.full((EP,), NP_ - 1, jnp.int32).at[:E].set(dst).reshape(EP // CH, CH)
    srcb = jnp.stack([srcp, srcp + NP_]).reshape(NC * (EP // CH), CH)
    dstd = jnp.concatenate([dstp, dstp], axis=0)
    srcp = srcp.reshape(EP // CH, CH)
    batchp = jnp.full((NS, RBP, CH), G, jnp.int32).at[:, :RB].set(
        jnp.full((NP_,), G, jnp.int32).at[:N].set(batch)
        .reshape(NS, RB, CH)).reshape(NS * RBP, CH)
    x_pad = jnp.zeros((NP_, D), jnp.float32).at[:N].set(x)
    zeros128 = jnp.zeros((CH, CH), jnp.float32)

    w0_cat = jnp.moveaxis(W0, 0, 1).reshape(D, T * H)
    b0_cat = b0.reshape(NC, 1, D)
    w1_quad, b1_cat = _quad(W1), b1.reshape(NC, 1, D)
    w2_quad, b2_cat = _quad(W2), b2.reshape(NC, 1, D)

    agg0p = _sc_agg_x(x_pad, srcp, dstp, zeros128)
    h1 = _tc1(x_pad, agg0p, w0_cat, b0_cat)
    h1f = h1.reshape(NC * NP_, D)
    agg1, r1 = _sc_agg_h(h1f, srcb, dstd, batchp, zeros128)
    h2 = _tc_mlp(h1, agg1, w1_quad, b1_cat)
    h2f = h2.reshape(NC * NP_, D)
    agg2, r2 = _sc_agg_h(h2f, srcb, dstd, batchp, zeros128)
    h3 = _tc_mlp(h2, agg2, w2_quad, b2_cat)
    r3 = _sc_readout(h3.reshape(NC * NP_, D), batchp, zeros128)

    out = jnp.concatenate([_fold(r1), _fold(r2), _fold(r3)], axis=2)
    return out.at[:, 0, :].add(bias[0, 0])
